# baseline (speedup 1.0000x reference)
import functools

import jax
import jax.numpy as jnp
from jax import lax
from jax.experimental import pallas as pl
from jax.experimental.pallas import tpu as pltpu
from jax.experimental.pallas import tpu_sc as plsc

DIM = 2048
HIDDEN = 1024
N_EXP = 16
TOPK = 2
BLK = 128
T = 4096
P = T * TOPK
PADROWS = P + N_EXP * BLK
G_E = PADROWS // BLK
TB_R = 512
TB_S = 256
TB_C = 512

SC_CORES = 2
SC_SUBCORES = 16
SC_WORKERS = SC_CORES * SC_SUBCORES
GATHER_CHUNK = 16


def _router_body(x_ref, gw_ref, w_ref, e_ref):
    xb = x_ref[...]
    logits = lax.dot_general(xb, gw_ref[...], (((1,), (1,)), ((), ())),
                             preferred_element_type=jnp.float32)
    iota = lax.broadcasted_iota(jnp.int32, logits.shape, 1)
    m1 = jnp.max(logits, axis=1, keepdims=True)
    e1 = jnp.min(jnp.where(logits == m1, iota, N_EXP), axis=1, keepdims=True)
    masked = jnp.where(iota == e1, -jnp.inf, logits)
    m2 = jnp.max(masked, axis=1, keepdims=True)
    e2 = jnp.min(jnp.where(masked == m2, iota, N_EXP), axis=1, keepdims=True)
    a2 = jnp.exp(m2 - m1)
    s = 1.0 + a2
    w_ref[:, 0:1] = 1.0 / s
    w_ref[:, 1:2] = a2 / s
    e_ref[:, 0:1] = e1
    e_ref[:, 1:2] = e2


def _router(x_flat, gate_w):
    return pl.pallas_call(
        _router_body,
        grid=(T // TB_R,),
        in_specs=[
            pl.BlockSpec((TB_R, DIM), lambda i: (i, 0)),
            pl.BlockSpec((N_EXP, DIM), lambda i: (0, 0)),
        ],
        out_specs=[
            pl.BlockSpec((TB_R, 128), lambda i: (i, 0)),
            pl.BlockSpec((TB_R, 128), lambda i: (i, 0)),
        ],
        out_shape=[
            jax.ShapeDtypeStruct((T, 128), jnp.float32),
            jax.ShapeDtypeStruct((T, 128), jnp.int32),
        ],
    )(x_flat, gate_w)


def _sc_gather(table, idx, n_rows, dim):
    rpw = n_rows // SC_WORKERS
    nchunk = rpw // GATHER_CHUNK
    mesh = plsc.VectorSubcoreMesh(core_axis_name="c", subcore_axis_name="s")

    @functools.partial(
        pl.kernel,
        out_type=jax.ShapeDtypeStruct((n_rows, dim), jnp.float32),
        mesh=mesh,
        scratch_types=[
            pltpu.VMEM((rpw,), jnp.int32),
            pltpu.VMEM((GATHER_CHUNK, dim), jnp.float32),
            pltpu.VMEM((GATHER_CHUNK, dim), jnp.float32),
            pltpu.SemaphoreType.DMA,
            pltpu.SemaphoreType.DMA,
            pltpu.SemaphoreType.DMA,
            pltpu.SemaphoreType.DMA,
        ],
    )
    def k(table_hbm, idx_hbm, out_hbm, idx_v, buf0, buf1, sg0, sg1, sw0, sw1):
        wid = lax.axis_index("s") * SC_CORES + lax.axis_index("c")
        base = wid * rpw
        pltpu.sync_copy(idx_hbm.at[pl.ds(base, rpw)], idx_v)
        bufs = (buf0, buf1)
        sgs = (sg0, sg1)
        sws = (sw0, sw1)

        def gather(c):
            off = c * GATHER_CHUNK
            return pltpu.async_copy(
                table_hbm.at[idx_v.at[pl.ds(off, GATHER_CHUNK)]],
                bufs[c % 2], sgs[c % 2])

        def write(c):
            off = c * GATHER_CHUNK
            return pltpu.async_copy(
                bufs[c % 2], out_hbm.at[pl.ds(base + off, GATHER_CHUNK)],
                sws[c % 2])

        pend_g = {0: gather(0)}
        pend_w = {}
        for c in range(nchunk):
            if c + 1 < nchunk:
                if c - 1 >= 0:
                    pend_w.pop(c - 1).wait()
                pend_g[c + 1] = gather(c + 1)
            pend_g.pop(c).wait()
            pend_w[c] = write(c)
        for c in list(pend_w):
            pend_w.pop(c).wait()

    return k(table, idx)


def _grouped_body(be_ref, vl_ref, xg_ref, w1_ref, w3_ref, w2_ref, out_ref,
                  w1s):
    g = pl.program_id(0)
    changed = jnp.logical_or(
        jnp.logical_or(g == 0, g == G_E // 2),
        be_ref[g] != be_ref[jnp.maximum(g - 1, 0)])

    @pl.when(changed)
    def _():
        w1s[...] = w1_ref[0].astype(jnp.bfloat16)

    @pl.when(vl_ref[g] == 1)
    def _():
        xb = xg_ref[...].astype(jnp.bfloat16)
        a = lax.dot_general(xb, w1s[...], (((1,), (1,)), ((), ())),
                            preferred_element_type=jnp.float32)
        b = lax.dot_general(xb, w3_ref[0].astype(jnp.bfloat16),
                            (((1,), (1,)), ((), ())),
                            preferred_element_type=jnp.float32)
        h = (jax.nn.silu(a) * b).astype(jnp.bfloat16)
        out_ref[...] = lax.dot_general(h, w2_ref[0].astype(jnp.bfloat16),
                                       (((1,), (1,)), ((), ())),
                                       preferred_element_type=jnp.float32)

    @pl.when(vl_ref[g] == 0)
    def _():
        out_ref[...] = jnp.zeros_like(out_ref)


def _grouped_gemm(block_expert, valid, xg, w1, w3, w2):
    grid_spec = pltpu.PrefetchScalarGridSpec(
        num_scalar_prefetch=2,
        grid=(G_E,),
        in_specs=[
            pl.BlockSpec((BLK, DIM), lambda g, be, vl: (g, 0)),
            pl.BlockSpec((1, HIDDEN, DIM), lambda g, be, vl: (be[g], 0, 0)),
            pl.BlockSpec((1, HIDDEN, DIM), lambda g, be, vl: (be[g], 0, 0)),
            pl.BlockSpec((1, DIM, HIDDEN), lambda g, be, vl: (be[g], 0, 0)),
        ],
        out_specs=pl.BlockSpec((BLK, DIM), lambda g, be, vl: (g, 0)),
        scratch_shapes=[
            pltpu.VMEM((HIDDEN, DIM), jnp.bfloat16),
        ],
    )
    return pl.pallas_call(
        _grouped_body,
        grid_spec=grid_spec,
        out_shape=jax.ShapeDtypeStruct((PADROWS, DIM), jnp.float32),
        compiler_params=pltpu.CompilerParams(
            dimension_semantics=("parallel",)),
    )(block_expert, valid, xg, w1, w3, w2)


def _shared_body(x_ref, sw1_ref, sw3_ref, sw2_ref, out_ref, w1s, w3s, w2s):
    i = pl.program_id(0)
    nb = pl.num_programs(0)

    @pl.when(jnp.logical_or(i == 0, i == nb // 2))
    def _():
        w1s[...] = sw1_ref[...].astype(jnp.bfloat16)
        w3s[...] = sw3_ref[...].astype(jnp.bfloat16)
        w2s[...] = sw2_ref[...].astype(jnp.bfloat16)

    xb = x_ref[...].astype(jnp.bfloat16)
    a = lax.dot_general(xb, w1s[...], (((1,), (1,)), ((), ())),
                        preferred_element_type=jnp.float32)
    b = lax.dot_general(xb, w3s[...], (((1,), (1,)), ((), ())),
                        preferred_element_type=jnp.float32)
    h = (jax.nn.silu(a) * b).astype(jnp.bfloat16)
    out_ref[...] = lax.dot_general(h, w2s[...], (((1,), (1,)), ((), ())),
                                   preferred_element_type=jnp.float32)


def _shared_expert(x_flat, sw1, sw3, sw2):
    return pl.pallas_call(
        _shared_body,
        grid=(T // TB_S,),
        in_specs=[
            pl.BlockSpec((TB_S, DIM), lambda i: (i, 0)),
            pl.BlockSpec((HIDDEN, DIM), lambda i: (0, 0)),
            pl.BlockSpec((HIDDEN, DIM), lambda i: (0, 0)),
            pl.BlockSpec((DIM, HIDDEN), lambda i: (0, 0)),
        ],
        out_specs=pl.BlockSpec((TB_S, DIM), lambda i: (i, 0)),
        out_shape=jax.ShapeDtypeStruct((T, DIM), jnp.float32),
        scratch_shapes=[
            pltpu.VMEM((HIDDEN, DIM), jnp.bfloat16),
            pltpu.VMEM((HIDDEN, DIM), jnp.bfloat16),
            pltpu.VMEM((DIM, HIDDEN), jnp.bfloat16),
        ],
        compiler_params=pltpu.CompilerParams(
            dimension_semantics=("parallel",)),
    )(x_flat, sw1, sw3, sw2)


def _combine_body(y0_ref, y1_ref, w_ref, sh_ref, out_ref):
    w0 = w_ref[:, 0:1]
    w1 = w_ref[:, 1:2]
    out_ref[...] = y0_ref[...] * w0 + y1_ref[...] * w1 + sh_ref[...]


def _combine(y2, w_out, shared):
    nb = T // TB_C
    return pl.pallas_call(
        _combine_body,
        grid=(nb,),
        in_specs=[
            pl.BlockSpec((TB_C, DIM), lambda i: (i, 0)),
            pl.BlockSpec((TB_C, DIM), lambda i, _nb=nb: (i + _nb, 0)),
            pl.BlockSpec((TB_C, 128), lambda i: (i, 0)),
            pl.BlockSpec((TB_C, DIM), lambda i: (i, 0)),
        ],
        out_specs=pl.BlockSpec((TB_C, DIM), lambda i: (i, 0)),
        out_shape=jax.ShapeDtypeStruct((T, DIM), jnp.float32),
    )(y2, y2, w_out, shared)


def kernel(x, gate_w, w1, w3, w2, sw1, sw3, sw2):
    bsz, seq, dim = x.shape
    x_flat = x.reshape(-1, dim)

    w_out, e_out = _router(x_flat, gate_w)

    e_p = e_out[:, :TOPK].reshape(-1)
    oh = (e_p[:, None] == jnp.arange(N_EXP)[None, :]).astype(jnp.float32)
    ohb = oh.reshape(P // BLK, BLK, N_EXP)
    bs = ohb.sum(axis=1)
    excl_blk = jnp.cumsum(bs, axis=0) - bs
    tri = jnp.tril(jnp.ones((BLK, BLK), jnp.float32))
    within = jax.lax.dot_general(
        tri, ohb, (((1,), (1,)), ((), ())))
    cum_incl = within.transpose(1, 0, 2) + excl_blk[:, None, :]
    rank = (cum_incl.reshape(P, N_EXP) * oh).sum(axis=1) - 1.0
    counts = bs.sum(axis=0)
    padded = jnp.ceil(counts / BLK) * BLK
    pad_start = jnp.cumsum(padded) - padded
    row_p = (oh @ pad_start + rank).astype(jnp.int32)
    src_tok = jnp.zeros((PADROWS,), jnp.int32).at[row_p].set(
        jnp.arange(P, dtype=jnp.int32) // TOPK)
    blk_cum = jnp.cumsum(padded) * (1.0 / BLK)
    gids = jnp.arange(G_E, dtype=jnp.float32)
    be = (gids[:, None] >= blk_cum[None, :]).sum(axis=1).astype(jnp.int32)
    valid = (gids < blk_cum[-1]).astype(jnp.int32)
    block_expert = jnp.minimum(be, N_EXP - 1)

    xg = _sc_gather(x_flat, src_tok, PADROWS, dim)

    go = _grouped_gemm(block_expert, valid, xg, w1, w3, w2)
    shared = _shared_expert(x_flat, sw1, sw3, sw2)

    idx2 = jnp.concatenate([row_p[0::TOPK], row_p[1::TOPK]])
    y2 = _sc_gather(go, idx2, P, dim)

    out = _combine(y2, w_out, shared)
    return out.reshape(bsz, seq, dim)


# speedup vs baseline: 1.0655x; 1.0655x over previous
import functools

import jax
import jax.numpy as jnp
from jax import lax
from jax.experimental import pallas as pl
from jax.experimental.pallas import tpu as pltpu
from jax.experimental.pallas import tpu_sc as plsc

DIM = 2048
HIDDEN = 1024
N_EXP = 16
TOPK = 2
BLK = 128
T = 4096
P = T * TOPK
PADROWS = P + N_EXP * BLK
G_E = PADROWS // BLK
TB_R = 512
TB_S = 256
TB_C = 512

SC_CORES = 2
SC_SUBCORES = 16
SC_WORKERS = SC_CORES * SC_SUBCORES
GATHER_CHUNK = 16


def _router_body(x_ref, gw_ref, w_ref, e_ref):
    xb = x_ref[...]
    logits = lax.dot_general(xb, gw_ref[...], (((1,), (1,)), ((), ())),
                             preferred_element_type=jnp.float32)
    iota = lax.broadcasted_iota(jnp.int32, logits.shape, 1)
    m1 = jnp.max(logits, axis=1, keepdims=True)
    e1 = jnp.min(jnp.where(logits == m1, iota, N_EXP), axis=1, keepdims=True)
    masked = jnp.where(iota == e1, -jnp.inf, logits)
    m2 = jnp.max(masked, axis=1, keepdims=True)
    e2 = jnp.min(jnp.where(masked == m2, iota, N_EXP), axis=1, keepdims=True)
    a2 = jnp.exp(m2 - m1)
    s = 1.0 + a2
    w_ref[:, 0:1] = 1.0 / s
    w_ref[:, 1:2] = a2 / s
    e_ref[:, 0:1] = e1
    e_ref[:, 1:2] = e2


def _router(x_flat, gate_w):
    return pl.pallas_call(
        _router_body,
        grid=(T // TB_R,),
        in_specs=[
            pl.BlockSpec((TB_R, DIM), lambda i: (i, 0)),
            pl.BlockSpec((N_EXP, DIM), lambda i: (0, 0)),
        ],
        out_specs=[
            pl.BlockSpec((TB_R, 128), lambda i: (i, 0)),
            pl.BlockSpec((TB_R, 128), lambda i: (i, 0)),
        ],
        out_shape=[
            jax.ShapeDtypeStruct((T, 128), jnp.float32),
            jax.ShapeDtypeStruct((T, 128), jnp.int32),
        ],
    )(x_flat, gate_w)


def _sc_gather(table, idx, n_rows, dim):
    rpw = n_rows // SC_WORKERS
    nchunk = rpw // GATHER_CHUNK
    mesh = plsc.VectorSubcoreMesh(core_axis_name="c", subcore_axis_name="s")

    @functools.partial(
        pl.kernel,
        out_type=jax.ShapeDtypeStruct((n_rows, dim), jnp.float32),
        mesh=mesh,
        scratch_types=[
            pltpu.VMEM((rpw,), jnp.int32),
            pltpu.VMEM((GATHER_CHUNK, dim), jnp.float32),
            pltpu.VMEM((GATHER_CHUNK, dim), jnp.float32),
            pltpu.SemaphoreType.DMA,
            pltpu.SemaphoreType.DMA,
            pltpu.SemaphoreType.DMA,
            pltpu.SemaphoreType.DMA,
        ],
    )
    def k(table_hbm, idx_hbm, out_hbm, idx_v, buf0, buf1, sg0, sg1, sw0, sw1):
        wid = lax.axis_index("s") * SC_CORES + lax.axis_index("c")
        base = wid * rpw
        pltpu.sync_copy(idx_hbm.at[pl.ds(base, rpw)], idx_v)
        bufs = (buf0, buf1)
        sgs = (sg0, sg1)
        sws = (sw0, sw1)

        def gather(c):
            off = c * GATHER_CHUNK
            return pltpu.async_copy(
                table_hbm.at[idx_v.at[pl.ds(off, GATHER_CHUNK)]],
                bufs[c % 2], sgs[c % 2])

        def write(c):
            off = c * GATHER_CHUNK
            return pltpu.async_copy(
                bufs[c % 2], out_hbm.at[pl.ds(base + off, GATHER_CHUNK)],
                sws[c % 2])

        pend_g = {0: gather(0)}
        pend_w = {}
        for c in range(nchunk):
            if c + 1 < nchunk:
                if c - 1 >= 0:
                    pend_w.pop(c - 1).wait()
                pend_g[c + 1] = gather(c + 1)
            pend_g.pop(c).wait()
            pend_w[c] = write(c)
        for c in list(pend_w):
            pend_w.pop(c).wait()

    return k(table, idx)


def _grouped_body(*refs):
    be_ref, vl_ref, xg_ref, w1_ref, w3_ref, w2_ref = refs[:6]
    out_ref = refs[-2]
    w1s = refs[-1]
    g = pl.program_id(0)
    changed = jnp.logical_or(
        g == 0, be_ref[g] != be_ref[jnp.maximum(g - 1, 0)])

    @pl.when(changed)
    def _():
        w1s[...] = w1_ref[0].astype(jnp.bfloat16)

    @pl.when(vl_ref[g] == 1)
    def _():
        xb = xg_ref[...].astype(jnp.bfloat16)
        a = lax.dot_general(xb, w1s[...], (((1,), (1,)), ((), ())),
                            preferred_element_type=jnp.float32)
        b = lax.dot_general(xb, w3_ref[0].astype(jnp.bfloat16),
                            (((1,), (1,)), ((), ())),
                            preferred_element_type=jnp.float32)
        h = (jax.nn.silu(a) * b).astype(jnp.bfloat16)
        out_ref[...] = lax.dot_general(h, w2_ref[0].astype(jnp.bfloat16),
                                       (((1,), (1,)), ((), ())),
                                       preferred_element_type=jnp.float32)

    @pl.when(vl_ref[g] == 0)
    def _():
        out_ref[...] = jnp.zeros_like(out_ref)


def _grouped_gemm(block_expert, valid, xg, w1, w3, w2, prev, half):
    gh = G_E // 2
    off = half * gh

    in_specs = [
        pl.BlockSpec((BLK, DIM), lambda g, be, vl: (g, 0)),
        pl.BlockSpec((1, HIDDEN, DIM), lambda g, be, vl: (be[g], 0, 0)),
        pl.BlockSpec((1, HIDDEN, DIM), lambda g, be, vl: (be[g], 0, 0)),
        pl.BlockSpec((1, DIM, HIDDEN), lambda g, be, vl: (be[g], 0, 0)),
    ]
    args = [block_expert, valid, xg, w1, w3, w2]
    aliases = {}
    if half == 1:
        in_specs.append(pl.BlockSpec((BLK, DIM), lambda g, be, vl: (0, 0)))
        args.append(prev)
        aliases = {6: 0}

    grid_spec = pltpu.PrefetchScalarGridSpec(
        num_scalar_prefetch=2,
        grid=(gh,),
        in_specs=in_specs,
        out_specs=pl.BlockSpec((BLK, DIM),
                               lambda g, be, vl, _o=off: (g + _o, 0)),
        scratch_shapes=[
            pltpu.VMEM((HIDDEN, DIM), jnp.bfloat16),
        ],
    )
    return pl.pallas_call(
        _grouped_body,
        grid_spec=grid_spec,
        out_shape=jax.ShapeDtypeStruct((PADROWS, DIM), jnp.float32),
        input_output_aliases=aliases,
        compiler_params=pltpu.CompilerParams(
            dimension_semantics=("arbitrary",)),
    )(*args)


def _shared_body(x_ref, sw1_ref, sw3_ref, sw2_ref, out_ref, w1s, w3s, w2s):
    i = pl.program_id(0)
    nb = pl.num_programs(0)

    @pl.when(jnp.logical_or(i == 0, i == nb // 2))
    def _():
        w1s[...] = sw1_ref[...].astype(jnp.bfloat16)
        w3s[...] = sw3_ref[...].astype(jnp.bfloat16)
        w2s[...] = sw2_ref[...].astype(jnp.bfloat16)

    xb = x_ref[...].astype(jnp.bfloat16)
    a = lax.dot_general(xb, w1s[...], (((1,), (1,)), ((), ())),
                        preferred_element_type=jnp.float32)
    b = lax.dot_general(xb, w3s[...], (((1,), (1,)), ((), ())),
                        preferred_element_type=jnp.float32)
    h = (jax.nn.silu(a) * b).astype(jnp.bfloat16)
    out_ref[...] = lax.dot_general(h, w2s[...], (((1,), (1,)), ((), ())),
                                   preferred_element_type=jnp.float32)


def _shared_expert(x_flat, sw1, sw3, sw2):
    return pl.pallas_call(
        _shared_body,
        grid=(T // TB_S,),
        in_specs=[
            pl.BlockSpec((TB_S, DIM), lambda i: (i, 0)),
            pl.BlockSpec((HIDDEN, DIM), lambda i: (0, 0)),
            pl.BlockSpec((HIDDEN, DIM), lambda i: (0, 0)),
            pl.BlockSpec((DIM, HIDDEN), lambda i: (0, 0)),
        ],
        out_specs=pl.BlockSpec((TB_S, DIM), lambda i: (i, 0)),
        out_shape=jax.ShapeDtypeStruct((T, DIM), jnp.float32),
        scratch_shapes=[
            pltpu.VMEM((HIDDEN, DIM), jnp.bfloat16),
            pltpu.VMEM((HIDDEN, DIM), jnp.bfloat16),
            pltpu.VMEM((DIM, HIDDEN), jnp.bfloat16),
        ],
        compiler_params=pltpu.CompilerParams(
            dimension_semantics=("parallel",)),
    )(x_flat, sw1, sw3, sw2)


def _combine_body(y0_ref, y1_ref, w_ref, sh_ref, out_ref):
    w0 = w_ref[:, 0:1]
    w1 = w_ref[:, 1:2]
    out_ref[...] = y0_ref[...] * w0 + y1_ref[...] * w1 + sh_ref[...]


def _combine(y2, w_out, shared):
    nb = T // TB_C
    return pl.pallas_call(
        _combine_body,
        grid=(nb,),
        in_specs=[
            pl.BlockSpec((TB_C, DIM), lambda i: (i, 0)),
            pl.BlockSpec((TB_C, DIM), lambda i, _nb=nb: (i + _nb, 0)),
            pl.BlockSpec((TB_C, 128), lambda i: (i, 0)),
            pl.BlockSpec((TB_C, DIM), lambda i: (i, 0)),
        ],
        out_specs=pl.BlockSpec((TB_C, DIM), lambda i: (i, 0)),
        out_shape=jax.ShapeDtypeStruct((T, DIM), jnp.float32),
    )(y2, y2, w_out, shared)


def kernel(x, gate_w, w1, w3, w2, sw1, sw3, sw2):
    bsz, seq, dim = x.shape
    x_flat = x.reshape(-1, dim)

    w_out, e_out = _router(x_flat, gate_w)

    e_p = e_out[:, :TOPK].reshape(-1)
    oh = (e_p[:, None] == jnp.arange(N_EXP)[None, :]).astype(jnp.float32)
    ohb = oh.reshape(P // BLK, BLK, N_EXP)
    bs = ohb.sum(axis=1)
    excl_blk = jnp.cumsum(bs, axis=0) - bs
    tri = jnp.tril(jnp.ones((BLK, BLK), jnp.float32))
    within = jax.lax.dot_general(
        tri, ohb, (((1,), (1,)), ((), ())))
    cum_incl = within.transpose(1, 0, 2) + excl_blk[:, None, :]
    rank = (cum_incl.reshape(P, N_EXP) * oh).sum(axis=1) - 1.0
    counts = bs.sum(axis=0)
    padded = jnp.ceil(counts / BLK) * BLK
    pad_start = jnp.cumsum(padded) - padded
    row_p = (oh @ pad_start + rank).astype(jnp.int32)
    src_tok = jnp.zeros((PADROWS,), jnp.int32).at[row_p].set(
        jnp.arange(P, dtype=jnp.int32) // TOPK)
    blk_cum = jnp.cumsum(padded) * (1.0 / BLK)
    gids = jnp.arange(G_E, dtype=jnp.float32)
    be = (gids[:, None] >= blk_cum[None, :]).sum(axis=1).astype(jnp.int32)
    valid = (gids < blk_cum[-1]).astype(jnp.int32)
    block_expert = jnp.minimum(be, N_EXP - 1)

    hrows = PADROWS // 2
    gh = G_E // 2
    xg_a = _sc_gather(x_flat, src_tok[:hrows], hrows, dim)
    xg_b = _sc_gather(x_flat, src_tok[hrows:], hrows, dim)
    go_a = _grouped_gemm(block_expert[:gh], valid[:gh], xg_a,
                         w1, w3, w2, None, 0)
    go = _grouped_gemm(block_expert[gh:], valid[gh:], xg_b,
                       w1, w3, w2, go_a, 1)
    shared = _shared_expert(x_flat, sw1, sw3, sw2)

    idx2 = jnp.concatenate([row_p[0::TOPK], row_p[1::TOPK]])
    y2 = _sc_gather(go, idx2, P, dim)

    out = _combine(y2, w_out, shared)
    return out.reshape(bsz, seq, dim)
